# baseline (speedup 1.0000x reference)
import jax
import jax.numpy as jnp
from jax.experimental import pallas as pl
from jax.experimental.pallas import tpu as pltpu

_B = 128
_N = 32768
_LANES = 128
_SUB = (_N * 4) // _LANES
_BS = 64
_CHUNKS = _SUB // _BS
_PPC = _BS * _LANES // 4
_IMAX = 2**31 - 1


def _body(w_ref, wk_ref, b_ref, z_ref, vals_ref, idx_ref):
    chunk = pl.program_id(0)

    @pl.when(chunk == 0)
    def _init():
        vals_ref[...] = jnp.full((_B, 3), -jnp.inf, jnp.float32)
        idx_ref[...] = jnp.full((_B, 3), _IMAX, jnp.int32)

    z = z_ref[...]
    def bf(v):
        return v.astype(jnp.bfloat16).astype(jnp.float32)

    t0 = bf(z + w_ref[0])
    t1 = bf(jnp.roll(z, -1, axis=2) + w_ref[1])
    t2 = bf(jnp.roll(z, -2, axis=2) + w_ref[2])
    t3 = bf(jnp.roll(z, -3, axis=2) + w_ref[3])

    y = []
    for k in range(5):
        g = ((t0 * wk_ref[k, 0] + t1 * wk_ref[k, 1])
             + (t2 * wk_ref[k, 2] + t3 * wk_ref[k, 3])) + b_ref[k]
        y.append(jnp.maximum(g, 0.0))
    s = ((y[0] + y[4]) + y[2]) + (y[1] + y[3])

    lane = jax.lax.broadcasted_iota(jnp.int32, (_B, _BS, _LANES), 2)
    sub = jax.lax.broadcasted_iota(jnp.int32, (_B, _BS, _LANES), 1)
    pos = chunk * _PPC + sub * (_LANES // 4) + lane // 4
    s = jnp.where(lane % 4 == 0, s, -jnp.inf)

    bvals, bidx = [], []
    for r in range(3):
        m = jnp.max(s, axis=(1, 2))
        hit = s == m[:, None, None]
        sel = jnp.min(jnp.where(hit, pos, _IMAX), axis=(1, 2))
        bvals.append(m)
        bidx.append(sel)
        if r < 2:
            s = jnp.where(pos == sel[:, None, None], -jnp.inf, s)

    av = jnp.concatenate([vals_ref[...], jnp.stack(bvals, axis=1)], axis=1)
    ai = jnp.concatenate([idx_ref[...], jnp.stack(bidx, axis=1)], axis=1)
    nv, ni = [], []
    for r in range(3):
        m = jnp.max(av, axis=1)
        sel = jnp.min(jnp.where(av == m[:, None], ai, _IMAX), axis=1)
        nv.append(m)
        ni.append(sel)
        av = jnp.where(ai == sel[:, None], -jnp.inf, av)
    vals_ref[...] = jnp.stack(nv, axis=1)
    idx_ref[...] = jnp.stack(ni, axis=1)


def kernel(x, W, b):
    z = x.reshape(_B, _SUB, _LANES)
    w = W.mean(axis=0)
    u = jax.lax.bitcast_convert_type(W, jnp.uint32)
    u = (u + jnp.uint32(0x7FFF) + ((u >> 16) & jnp.uint32(1))) & jnp.uint32(0xFFFF0000)
    W = jax.lax.bitcast_convert_type(u, jnp.float32)
    vals, idx = pl.pallas_call(
        _body,
        grid=(_CHUNKS,),
        in_specs=[
            pl.BlockSpec(memory_space=pltpu.SMEM),
            pl.BlockSpec(memory_space=pltpu.SMEM),
            pl.BlockSpec(memory_space=pltpu.SMEM),
            pl.BlockSpec((_B, _BS, _LANES), lambda i: (0, i, 0)),
        ],
        out_specs=[
            pl.BlockSpec((_B, 3), lambda i: (0, 0)),
            pl.BlockSpec((_B, 3), lambda i: (0, 0)),
        ],
        out_shape=[
            jax.ShapeDtypeStruct((_B, 3), jnp.float32),
            jax.ShapeDtypeStruct((_B, 3), jnp.int32),
        ],
        compiler_params=pltpu.CompilerParams(
            dimension_semantics=("arbitrary",)),
    )(w, W, b, z)
    return vals, idx


# speedup vs baseline: 1.1431x; 1.1431x over previous
import jax
import jax.numpy as jnp
from jax.experimental import pallas as pl
from jax.experimental.pallas import tpu as pltpu

_B = 128
_N = 32768
_LANES = 128
_SUB = (_N * 4) // _LANES
_BS = 64
_CHUNKS = _SUB // _BS
_QUADS = _BS // 4
_IMAX = 2**31 - 1
_NEG = float("-inf")


def _body(b_ref, aux_ref, lp_ref, z_ref, vals_ref, idx_ref,
          m0_ref, m1_ref, m2_ref, i0_ref, i1_ref, i2_ref):
    chunk = pl.program_id(0)

    @pl.when(chunk == 0)
    def _init():
        neg = jnp.full((_B, _LANES), _NEG, jnp.float32)
        zero = jnp.zeros((_B, _LANES), jnp.int32)
        m0_ref[...] = neg
        m1_ref[...] = neg
        m2_ref[...] = neg
        i0_ref[...] = zero
        i1_ref[...] = zero
        i2_ref[...] = zero

    z = z_ref[...]
    wpat = aux_ref[14]
    bpat = aux_ref[15]
    maskf = aux_ref[16]

    zb = (z + wpat).astype(jnp.bfloat16).astype(jnp.float32)
    zs = {d: jnp.roll(zb, -d, axis=2) for d in range(-3, 4) if d != 0}
    zs[0] = zb

    G = zs[-3] * aux_ref[0]
    g4 = zs[-3] * aux_ref[7]
    for d in range(-2, 4):
        G = G + zs[d] * aux_ref[3 + d]
        g4 = g4 + zs[d] * aux_ref[10 + d]
    Y = jnp.maximum(G + bpat, 0.0)
    y4 = jnp.maximum(g4 + b_ref[4], 0.0)
    u = Y + y4 * maskf
    s2 = u + jnp.roll(u, -2, axis=2)
    s = s2 + jnp.roll(s2, -1, axis=2)

    lane = jax.lax.broadcasted_iota(jnp.int32, (_B, _LANES), 1)
    m = lane % 4
    mm1, mm2, mm3 = m == 1, m == 2, m == 3

    for q in range(_QUADS):
        r0 = s[:, 4 * q, :]
        r1 = jnp.roll(s[:, 4 * q + 1, :], 1, axis=1)
        r2 = jnp.roll(s[:, 4 * q + 2, :], 2, axis=1)
        r3 = jnp.roll(s[:, 4 * q + 3, :], 3, axis=1)
        v = jnp.where(mm1, r1, jnp.where(mm2, r2, jnp.where(mm3, r3, r0)))
        step = chunk * _QUADS + q

        m0 = m0_ref[...]
        m1v = m1_ref[...]
        m2v = m2_ref[...]
        c0 = v > m0
        c1 = v > m1v
        c2 = v > m2v
        m2_ref[...] = jnp.where(c1, m1v, jnp.where(c2, v, m2v))
        m1_ref[...] = jnp.where(c0, m0, jnp.where(c1, v, m1v))
        m0_ref[...] = jnp.where(c0, v, m0)
        i0 = i0_ref[...]
        i1 = i1_ref[...]
        i2 = i2_ref[...]
        i2_ref[...] = jnp.where(c1, i1, jnp.where(c2, step, i2))
        i1_ref[...] = jnp.where(c0, i0, jnp.where(c1, step, i1))
        i0_ref[...] = jnp.where(c0, step, i0)

    @pl.when(chunk == _CHUNKS - 1)
    def _finalize():
        lp = lp_ref[0]
        cat = jnp.concatenate([m0_ref[...], m1_ref[...], m2_ref[...]], axis=1)
        pcat = jnp.concatenate(
            [i0_ref[...] * _LANES + lp, i1_ref[...] * _LANES + lp,
             i2_ref[...] * _LANES + lp], axis=1)
        nv, ni = [], []
        for r in range(3):
            mx = jnp.max(cat, axis=1)
            sel = jnp.min(jnp.where(cat == mx[:, None], pcat, _IMAX), axis=1)
            nv.append(mx)
            ni.append(sel)
            if r < 2:
                cat = jnp.where(pcat == sel[:, None], _NEG, cat)
        vals_ref[...] = jnp.stack(nv, axis=1)
        idx_ref[...] = jnp.stack(ni, axis=1)


def kernel(x, W, b):
    z = x.reshape(_B, _SUB, _LANES)
    w = W.mean(axis=0)
    u = jax.lax.bitcast_convert_type(W, jnp.uint32)
    u = (u + jnp.uint32(0x7FFF) + ((u >> 16) & jnp.uint32(1))) & jnp.uint32(0xFFFF0000)
    Wq = jax.lax.bitcast_convert_type(u, jnp.float32)

    lanes = jnp.arange(_LANES)
    ml = lanes % 4
    rows = []
    for base in (ml, jnp.full_like(ml, 4)):
        for d in range(-3, 4):
            c = ml + d
            valid = (c >= 0) & (c <= 3)
            rows.append(jnp.where(valid, Wq[base, jnp.clip(c, 0, 3)], 0.0))
    rows.append(w[ml])
    rows.append(b[jnp.minimum(ml, 4)])
    rows.append((ml == 0).astype(jnp.float32))
    aux = jnp.stack(rows, axis=0)
    lp = (ml * 32 + lanes // 4).astype(jnp.int32)[None, :]

    vals, idx = pl.pallas_call(
        _body,
        grid=(_CHUNKS,),
        in_specs=[
            pl.BlockSpec(memory_space=pltpu.SMEM),
            pl.BlockSpec((17, _LANES), lambda i: (0, 0)),
            pl.BlockSpec((1, _LANES), lambda i: (0, 0)),
            pl.BlockSpec((_B, _BS, _LANES), lambda i: (0, i, 0)),
        ],
        out_specs=[
            pl.BlockSpec((_B, 3), lambda i: (0, 0)),
            pl.BlockSpec((_B, 3), lambda i: (0, 0)),
        ],
        out_shape=[
            jax.ShapeDtypeStruct((_B, 3), jnp.float32),
            jax.ShapeDtypeStruct((_B, 3), jnp.int32),
        ],
        scratch_shapes=[
            pltpu.VMEM((_B, _LANES), jnp.float32),
            pltpu.VMEM((_B, _LANES), jnp.float32),
            pltpu.VMEM((_B, _LANES), jnp.float32),
            pltpu.VMEM((_B, _LANES), jnp.int32),
            pltpu.VMEM((_B, _LANES), jnp.int32),
            pltpu.VMEM((_B, _LANES), jnp.int32),
        ],
        compiler_params=pltpu.CompilerParams(
            dimension_semantics=("arbitrary",)),
    )(b, aux, lp, z)
    return vals, idx


# speedup vs baseline: 1.2732x; 1.1138x over previous
import jax
import jax.numpy as jnp
from jax.experimental import pallas as pl
from jax.experimental.pallas import tpu as pltpu

_B = 128
_N = 32768
_F = _N * 4
_CW = 8192
_CHUNKS = _F // _CW
_SPANS = _CW // 512
_LANES = 128
_IMAX = 2**31 - 1
_NEG = float("-inf")


def _body(b_ref, wq4_ref, aux_ref, lp_ref, z_ref, vals_ref, idx_ref,
          m0_ref, m1_ref, m2_ref, i0_ref, i1_ref, i2_ref):
    chunk = pl.program_id(0)

    @pl.when(chunk == 0)
    def _init():
        neg = jnp.full((_B, _LANES), _NEG, jnp.float32)
        zero = jnp.zeros((_B, _LANES), jnp.int32)
        m0_ref[...] = neg
        m1_ref[...] = neg
        m2_ref[...] = neg
        i0_ref[...] = zero
        i1_ref[...] = zero
        i2_ref[...] = zero

    z = z_ref[...]
    wpat = aux_ref[7]
    bpat = aux_ref[8]
    maskf = aux_ref[9]

    zb = (z + wpat).astype(jnp.bfloat16).astype(jnp.float32)
    zs = {d: jnp.roll(zb, -d, axis=1) for d in range(-3, 4) if d != 0}
    zs[0] = zb

    G = zs[-3] * aux_ref[0]
    for d in range(-2, 4):
        G = G + zs[d] * aux_ref[3 + d]
    g4 = zs[0] * wq4_ref[0]
    for c in range(1, 4):
        g4 = g4 + zs[c] * wq4_ref[c]
    Y = jnp.maximum(G + bpat, 0.0)
    y4 = jnp.maximum(g4 + b_ref[4], 0.0)
    u = Y + y4 * maskf
    s2 = u + jnp.roll(u, -2, axis=1)
    s = s2 + jnp.roll(s2, -1, axis=1)

    lane = jax.lax.broadcasted_iota(jnp.int32, (_B, _LANES), 1)
    m = lane % 4
    mm1, mm2, mm3 = m == 1, m == 2, m == 3

    for q in range(_SPANS):
        base = 512 * q
        r0 = s[:, base:base + 128]
        r1 = jnp.roll(s[:, base + 128:base + 256], 1, axis=1)
        r2 = jnp.roll(s[:, base + 256:base + 384], 2, axis=1)
        r3 = jnp.roll(s[:, base + 384:base + 512], 3, axis=1)
        v = jnp.where(mm1, r1, jnp.where(mm2, r2, jnp.where(mm3, r3, r0)))
        step = chunk * _SPANS + q

        m0 = m0_ref[...]
        m1v = m1_ref[...]
        m2v = m2_ref[...]
        c0 = v > m0
        c1 = v > m1v
        c2 = v > m2v
        m2_ref[...] = jnp.where(c1, m1v, jnp.where(c2, v, m2v))
        m1_ref[...] = jnp.where(c0, m0, jnp.where(c1, v, m1v))
        m0_ref[...] = jnp.where(c0, v, m0)
        i0 = i0_ref[...]
        i1 = i1_ref[...]
        i2 = i2_ref[...]
        i2_ref[...] = jnp.where(c1, i1, jnp.where(c2, step, i2))
        i1_ref[...] = jnp.where(c0, i0, jnp.where(c1, step, i1))
        i0_ref[...] = jnp.where(c0, step, i0)

    @pl.when(chunk == _CHUNKS - 1)
    def _finalize():
        lp = lp_ref[0]
        cat = jnp.concatenate([m0_ref[...], m1_ref[...], m2_ref[...]], axis=1)
        pcat = jnp.concatenate(
            [i0_ref[...] * _LANES + lp, i1_ref[...] * _LANES + lp,
             i2_ref[...] * _LANES + lp], axis=1)
        nv, ni = [], []
        for r in range(3):
            mx = jnp.max(cat, axis=1)
            sel = jnp.min(jnp.where(cat == mx[:, None], pcat, _IMAX), axis=1)
            nv.append(mx)
            ni.append(sel)
            if r < 2:
                cat = jnp.where(pcat == sel[:, None], _NEG, cat)
        vals_ref[...] = jnp.stack(nv, axis=1)
        idx_ref[...] = jnp.stack(ni, axis=1)


def kernel(x, W, b):
    z = x.reshape(_B, _F)
    w = W.mean(axis=0)
    u = jax.lax.bitcast_convert_type(W, jnp.uint32)
    u = (u + jnp.uint32(0x7FFF) + ((u >> 16) & jnp.uint32(1))) & jnp.uint32(0xFFFF0000)
    Wq = jax.lax.bitcast_convert_type(u, jnp.float32)

    lanes = jnp.arange(_LANES)
    ml = lanes % 4
    rows = []
    for d in range(-3, 4):
        c = ml + d
        valid = (c >= 0) & (c <= 3)
        rows.append(jnp.where(valid, Wq[ml, jnp.clip(c, 0, 3)], 0.0))
    rows.append(w[ml])
    rows.append(b[ml])
    rows.append((ml == 0).astype(jnp.float32))
    aux = jnp.tile(jnp.stack(rows, axis=0), (1, _CW // _LANES))
    lp = (ml * 32 + lanes // 4).astype(jnp.int32)[None, :]

    vals, idx = pl.pallas_call(
        _body,
        grid=(_CHUNKS,),
        in_specs=[
            pl.BlockSpec(memory_space=pltpu.SMEM),
            pl.BlockSpec(memory_space=pltpu.SMEM),
            pl.BlockSpec((10, _CW), lambda i: (0, 0)),
            pl.BlockSpec((1, _LANES), lambda i: (0, 0)),
            pl.BlockSpec((_B, _CW), lambda i: (0, i)),
        ],
        out_specs=[
            pl.BlockSpec((_B, 3), lambda i: (0, 0)),
            pl.BlockSpec((_B, 3), lambda i: (0, 0)),
        ],
        out_shape=[
            jax.ShapeDtypeStruct((_B, 3), jnp.float32),
            jax.ShapeDtypeStruct((_B, 3), jnp.int32),
        ],
        scratch_shapes=[
            pltpu.VMEM((_B, _LANES), jnp.float32),
            pltpu.VMEM((_B, _LANES), jnp.float32),
            pltpu.VMEM((_B, _LANES), jnp.float32),
            pltpu.VMEM((_B, _LANES), jnp.int32),
            pltpu.VMEM((_B, _LANES), jnp.int32),
            pltpu.VMEM((_B, _LANES), jnp.int32),
        ],
        compiler_params=pltpu.CompilerParams(
            dimension_semantics=("arbitrary",)),
    )(b, Wq[4], aux, lp, z)
    return vals, idx
